# baseline (speedup 1.0000x reference)
import jax
import jax.numpy as jnp
from jax.experimental import pallas as pl
from jax.experimental.pallas import tpu as pltpu

_BLOCK_ROWS = 512
_BUFFER_COUNT = 6


def _patch_first_rows(o_blk):
    blk = o_blk[0:8, :]
    rows = jax.lax.broadcasted_iota(jnp.int32, blk.shape, 0)
    cols = jax.lax.broadcasted_iota(jnp.int32, blk.shape, 1)
    blk = jnp.where(rows == 2, jnp.float32(1.0), blk)
    blk = jnp.where((rows == 1) & (cols == 1), jnp.float32(5.0), blk)
    o_blk[0:8, :] = blk


def _outer(x_hbm, o_hbm):
    n_rows, n_cols = x_hbm.shape

    def _inner(idx, x_blk, o_blk):
        (i,) = idx
        o_blk[...] = jnp.log(x_blk[...])

        @pl.when(i == 0)
        def _():
            _patch_first_rows(o_blk)

    in_spec = pl.BlockSpec(
        (_BLOCK_ROWS, n_cols),
        lambda i: (i, 0),
        pipeline_mode=pl.Buffered(buffer_count=_BUFFER_COUNT),
    )
    out_spec = pl.BlockSpec((_BLOCK_ROWS, n_cols), lambda i: (i, 0))
    pipe = pltpu.emit_pipeline(
        _inner,
        grid=(n_rows // _BLOCK_ROWS,),
        in_specs=[in_spec],
        out_specs=[out_spec],
        _explicit_indices=True,
    )
    pipe(x_hbm, o_hbm)


def kernel(x):
    n_rows, n_cols = x.shape
    return pl.pallas_call(
        _outer,
        in_specs=[pl.BlockSpec(memory_space=pl.ANY)],
        out_specs=pl.BlockSpec(memory_space=pl.ANY),
        out_shape=jax.ShapeDtypeStruct((n_rows, n_cols), x.dtype),
    )(x)


# speedup vs baseline: 1.0123x; 1.0123x over previous
import jax
import jax.numpy as jnp
from jax.experimental import pallas as pl
from jax.experimental.pallas import tpu as pltpu

_BLOCK_ROWS = 2048
_BUFFER_COUNT = 3


def _patch_first_rows(o_blk):
    blk = o_blk[0:8, :]
    rows = jax.lax.broadcasted_iota(jnp.int32, blk.shape, 0)
    cols = jax.lax.broadcasted_iota(jnp.int32, blk.shape, 1)
    blk = jnp.where(rows == 2, jnp.float32(1.0), blk)
    blk = jnp.where((rows == 1) & (cols == 1), jnp.float32(5.0), blk)
    o_blk[0:8, :] = blk


def _outer(x_hbm, o_hbm):
    n_rows, n_cols = x_hbm.shape

    def _inner(idx, x_blk, o_blk):
        (i,) = idx
        o_blk[...] = jnp.log(x_blk[...])

        @pl.when(i == 0)
        def _():
            _patch_first_rows(o_blk)

    in_spec = pl.BlockSpec(
        (_BLOCK_ROWS, n_cols),
        lambda i: (i, 0),
        pipeline_mode=pl.Buffered(buffer_count=_BUFFER_COUNT),
    )
    out_spec = pl.BlockSpec((_BLOCK_ROWS, n_cols), lambda i: (i, 0))
    pipe = pltpu.emit_pipeline(
        _inner,
        grid=(n_rows // _BLOCK_ROWS,),
        in_specs=[in_spec],
        out_specs=[out_spec],
        _explicit_indices=True,
    )
    pipe(x_hbm, o_hbm)


def kernel(x):
    n_rows, n_cols = x.shape
    return pl.pallas_call(
        _outer,
        in_specs=[pl.BlockSpec(memory_space=pl.ANY)],
        out_specs=pl.BlockSpec(memory_space=pl.ANY),
        out_shape=jax.ShapeDtypeStruct((n_rows, n_cols), x.dtype),
    )(x)
